# Initial kernel scaffold; baseline (speedup 1.0000x reference)
import jax
import jax.numpy as jnp
from jax.experimental import pallas as pl


def kernel(x, adj, W1, b1, W2, b2):
    raise NotImplementedError("write your pallas kernel here")


# speedup vs baseline: 4.6042x; 4.6042x over previous
import functools

import jax
import jax.numpy as jnp
from jax import lax
from jax.experimental import pallas as pl
from jax.experimental.pallas import tpu as pltpu
from jax.experimental.pallas import tpu_sc as plsc

N = 10000
D = 128
E = 320000

NC = 2
NS = 16
NW = NC * NS
EPW = E // NW
CHUNK = 80
NCHUNK = EPW // CHUNK
NPAD = 10240
AGG_PER_SUB = NPAD // NS
DEG_PER_SUB = NPAD // NS

ROWS_BLK = 1000


def _sc_aggregate(h, src, dst, zrows, zdeg, ones_c):
    mesh = plsc.VectorSubcoreMesh(
        core_axis_name="c", subcore_axis_name="s", num_cores=NC, num_subcores=NS
    )

    @functools.partial(
        pl.kernel,
        out_type=(
            jax.ShapeDtypeStruct((NC, NPAD, D), jnp.float32),
            jax.ShapeDtypeStruct((NC, NPAD), jnp.float32),
        ),
        mesh=mesh,
        scratch_types=(
            pltpu.VMEM_SHARED((NPAD, D), jnp.float32),
            pltpu.VMEM_SHARED((NPAD,), jnp.float32),
            pltpu.VMEM((CHUNK,), jnp.int32),
            pltpu.VMEM((CHUNK,), jnp.int32),
            pltpu.VMEM((CHUNK, D), jnp.float32),
            pltpu.VMEM((CHUNK,), jnp.float32),
            pltpu.SemaphoreType.DMA,
        ),
    )
    def k(h_hbm, src_hbm, dst_hbm, zrows_hbm, zdeg_hbm, ones_hbm,
          agg_out, deg_out, agg_sh, deg_sh, src_v, dst_v, rows_v, ones_v, sem):
        c = lax.axis_index("c")
        s = lax.axis_index("s")
        wid = c * NS + s
        base = wid * EPW

        pltpu.sync_copy(zrows_hbm, agg_sh.at[pl.ds(s * AGG_PER_SUB, AGG_PER_SUB)])
        pltpu.sync_copy(zdeg_hbm, deg_sh.at[pl.ds(s * DEG_PER_SUB, DEG_PER_SUB)])
        pltpu.sync_copy(ones_hbm, ones_v)
        plsc.subcore_barrier()

        def body(i, carry):
            off = base + i * CHUNK
            pltpu.sync_copy(src_hbm.at[pl.ds(off, CHUNK)], src_v)
            pltpu.sync_copy(dst_hbm.at[pl.ds(off, CHUNK)], dst_v)
            pltpu.async_copy(h_hbm.at[src_v], rows_v, sem).wait()
            pltpu.sync_copy(rows_v, agg_sh.at[dst_v], add=True)
            pltpu.sync_copy(ones_v, deg_sh.at[dst_v], add=True)
            return carry

        lax.fori_loop(0, NCHUNK, body, 0)
        plsc.subcore_barrier()

        pltpu.sync_copy(
            agg_sh.at[pl.ds(s * AGG_PER_SUB, AGG_PER_SUB)],
            agg_out.at[c, pl.ds(s * AGG_PER_SUB, AGG_PER_SUB)],
        )
        pltpu.sync_copy(
            deg_sh.at[pl.ds(s * DEG_PER_SUB, DEG_PER_SUB)],
            deg_out.at[c, pl.ds(s * DEG_PER_SUB, DEG_PER_SUB)],
        )

    return k(h, src, dst, zrows, zdeg, ones_c)


def _pre_body(x_ref, w_ref, b_ref, o_ref):
    x = x_ref[...]
    n = jnp.sqrt(jnp.sum(x * x, axis=1, keepdims=True))
    n = jnp.maximum(n, 1e-7)
    u = (jnp.arctan2(n, jnp.ones_like(n)) / n) * x
    o_ref[...] = (
        lax.dot(u, w_ref[...], preferred_element_type=jnp.float32) + b_ref[...]
    )


def _combine(h_ref, a_ref, d_ref):
    h = h_ref[...]
    agg = a_ref[0] + a_ref[1]
    deg = d_ref[:, 0] + d_ref[:, 1]
    agg = agg / jnp.maximum(deg, 1.0)[:, None]
    out = jnp.maximum(h + agg, 0.0)
    nrm = jnp.sqrt(jnp.sum(out * out, axis=1, keepdims=True))
    return out / (nrm + 1e-7)


def _mid_body(h_ref, a_ref, d_ref, w_ref, b_ref, o_ref):
    u = _combine(h_ref, a_ref, d_ref)
    o_ref[...] = (
        lax.dot(u, w_ref[...], preferred_element_type=jnp.float32) + b_ref[...]
    )


def _post_body(h_ref, a_ref, d_ref, o_ref):
    u = _combine(h_ref, a_ref, d_ref)
    n = jnp.sqrt(jnp.sum(u * u, axis=1, keepdims=True))
    n = jnp.maximum(n, 1e-7)
    t = jnp.clip(n, 0.0, 1.5)
    o_ref[...] = (jnp.tan(t) / n) * u


def _row_grid(nrows):
    return nrows // ROWS_BLK


_W_SPEC = pl.BlockSpec((D, D), lambda i: (0, 0))
_B_SPEC = pl.BlockSpec((1, D), lambda i: (0, 0))
_ROW_SPEC = pl.BlockSpec((ROWS_BLK, D), lambda i: (i, 0))
_AGG_SPEC = pl.BlockSpec((NC, ROWS_BLK, D), lambda i: (0, i, 0))
_DEG_SPEC = pl.BlockSpec((ROWS_BLK, NC), lambda i: (i, 0))


def _pre(x, w, b):
    return pl.pallas_call(
        _pre_body,
        grid=(_row_grid(N),),
        in_specs=[_ROW_SPEC, _W_SPEC, _B_SPEC],
        out_specs=_ROW_SPEC,
        out_shape=jax.ShapeDtypeStruct((N, D), jnp.float32),
    )(x, w, b)


def _mid(h, agg_part, deg_part, w, b):
    return pl.pallas_call(
        _mid_body,
        grid=(_row_grid(N),),
        in_specs=[_ROW_SPEC, _AGG_SPEC, _DEG_SPEC, _W_SPEC, _B_SPEC],
        out_specs=_ROW_SPEC,
        out_shape=jax.ShapeDtypeStruct((N, D), jnp.float32),
    )(h, agg_part, deg_part, w, b)


def _post(h, agg_part, deg_part):
    return pl.pallas_call(
        _post_body,
        grid=(_row_grid(N),),
        in_specs=[_ROW_SPEC, _AGG_SPEC, _DEG_SPEC],
        out_specs=_ROW_SPEC,
        out_shape=jax.ShapeDtypeStruct((N, D), jnp.float32),
    )(h, agg_part, deg_part)


def kernel(x, adj, W1, b1, W2, b2):
    zrows = jnp.zeros((AGG_PER_SUB, D), jnp.float32)
    zdeg = jnp.zeros((DEG_PER_SUB,), jnp.float32)
    ones_c = jnp.ones((CHUNK,), jnp.float32)
    b1r = b1.reshape(1, D)
    b2r = b2.reshape(1, D)

    h1 = _pre(x, W1, b1r)
    agg1, deg1 = _sc_aggregate(h1, adj[0, 0], adj[0, 1], zrows, zdeg, ones_c)
    h2 = _mid(h1, agg1[:, :N], deg1[:, :N].T, W2, b2r)
    agg2, deg2 = _sc_aggregate(h2, adj[1, 0], adj[1, 1], zrows, zdeg, ones_c)
    return _post(h2, agg2[:, :N], deg2[:, :N].T)
